# Initial kernel scaffold; baseline (speedup 1.0000x reference)
import jax
import jax.numpy as jnp
from jax.experimental import pallas as pl


def kernel(Gu, Gi, W0, a_src0, a_dst0, b0, W1, a_src1, a_dst1, b1, edge_index, user, pos):
    raise NotImplementedError("write your pallas kernel here")


# speedup vs baseline: 21.6914x; 21.6914x over previous
import functools

import jax
import jax.numpy as jnp
from jax import lax
from jax.experimental import pallas as pl
from jax.experimental.pallas import tpu as pltpu
from jax.experimental.pallas import tpu_sc as plsc

N_USERS = 25000
N_ITEMS = 25000
N = N_USERS + N_ITEMS
E = 800000
K = 64
BATCH = 4096

NS = 16
NC = 2
C_EDGE = 256
KSUB = C_EDGE // 128
N_CHUNKS = E // C_EDGE
CHUNK_ITERS = -(-N_CHUNKS // NS)
ZCH = 200
NZ = N // ZCH
N2 = 50048
DCH = 2176
ND = N2 // DCH

BLK = 5000
GRID_I = N // BLK


def _prep0_body(x_ref, w_ref, a2_ref, hlo_ref, hhi_ref, al_ref):
    x = x_ref[...]
    hlo_ref[...] = jnp.dot(x, w_ref[0], preferred_element_type=jnp.float32)
    hhi_ref[...] = jnp.dot(x, w_ref[1], preferred_element_type=jnp.float32)
    al_ref[...] = jnp.dot(x, a2_ref[...], preferred_element_type=jnp.float32)


_PREP_OUT_SPECS = None


def _prep_specs():
    in_specs = [
        pl.BlockSpec((BLK, K), lambda i: (i, 0)),
        pl.BlockSpec((2, K, 32), lambda i: (0, 0, 0)),
        pl.BlockSpec((K, 64), lambda i: (0, 0)),
    ]
    out_specs = [
        pl.BlockSpec((BLK, 32), lambda i: (i, 0)),
        pl.BlockSpec((BLK, 32), lambda i: (i, 0)),
        pl.BlockSpec((BLK, 64), lambda i: (i, 0)),
    ]
    out_shape = [
        jax.ShapeDtypeStruct((N, 32), jnp.float32),
        jax.ShapeDtypeStruct((N, 32), jnp.float32),
        jax.ShapeDtypeStruct((N, 64), jnp.float32),
    ]
    return in_specs, out_specs, out_shape


def _tc_prep0(x, W2, A2p):
    in_specs, out_specs, out_shape = _prep_specs()
    return pl.pallas_call(
        _prep0_body, grid=(GRID_I,), in_specs=in_specs,
        out_specs=out_specs, out_shape=out_shape,
    )(x, W2, A2p)


def _prep1_body(agg_ref, den_ref, b_ref, w_ref, a2_ref,
                hlo_ref, hhi_ref, al_ref):
    d = den_ref[...]
    x = agg_ref[...] / (d + 1e-16) + b_ref[...]
    x = jnp.where(x > 0, x, jnp.exp(x) - 1.0)
    hlo_ref[...] = jnp.dot(x, w_ref[0], preferred_element_type=jnp.float32)
    hhi_ref[...] = jnp.dot(x, w_ref[1], preferred_element_type=jnp.float32)
    al_ref[...] = jnp.dot(x, a2_ref[...], preferred_element_type=jnp.float32)


def _tc_prep1(agg, den2d, brow, W2, A2p):
    in_specs, out_specs, out_shape = _prep_specs()
    in_specs = [
        in_specs[0],
        pl.BlockSpec((BLK, 1), lambda i: (i, 0)),
        pl.BlockSpec((1, K), lambda i: (0, 0)),
        in_specs[1],
        in_specs[2],
    ]
    return pl.pallas_call(
        _prep1_body, grid=(GRID_I,), in_specs=in_specs,
        out_specs=out_specs, out_shape=out_shape,
    )(agg, den2d, brow, W2, A2p)


def _final_body(agg_ref, den_ref, b_ref, out_ref):
    d = den_ref[...]
    out_ref[...] = agg_ref[...] / (d + 1e-16) + b_ref[...]


def _tc_final(agg, den2d, brow):
    return pl.pallas_call(
        _final_body,
        grid=(GRID_I,),
        in_specs=[
            pl.BlockSpec((BLK, K), lambda i: (i, 0)),
            pl.BlockSpec((BLK, 1), lambda i: (i, 0)),
            pl.BlockSpec((1, K), lambda i: (0, 0)),
        ],
        out_specs=pl.BlockSpec((BLK, K), lambda i: (i, 0)),
        out_shape=jax.ShapeDtypeStruct((N, K), jnp.float32),
    )(agg, den2d, brow)


_MESH = plsc.VectorSubcoreMesh(core_axis_name="c", subcore_axis_name="s")


def _edge_body(as_hbm, ad_hbm, src_hbm, dst_hbm, hlo_hbm, hhi_hbm,
               agg_hbm, den_hbm,
               src2_v, dst2_v, asv_v, adv_v, w_v, rows_v, zden_v,
               as_sp, ad_sp, acc_sp, den_sp, sem):
    c = lax.axis_index("c")
    s = lax.axis_index("s")

    z16 = jnp.zeros((16,), jnp.float32)

    def _zrow(r, _):
        rows_v[r, pl.ds(0, 16)] = z16
        rows_v[r, pl.ds(16, 16)] = z16
        return 0
    lax.fori_loop(0, C_EDGE, _zrow, 0)

    def _zacc(j, _):
        cid = j * NS + s
        @pl.when(cid < NZ)
        def _():
            pltpu.sync_copy(rows_v.at[pl.ds(0, ZCH)],
                            acc_sp.at[pl.ds(cid * ZCH, ZCH)])
        return 0
    lax.fori_loop(0, -(-NZ // NS), _zacc, 0)

    @pl.when(s == 0)
    def _():
        pltpu.sync_copy(as_hbm, as_sp)

    @pl.when(s == 1)
    def _():
        pltpu.sync_copy(ad_hbm, ad_sp)

    @pl.when(c == 0)
    def _():
        for i in range(DCH // 16):
            zden_v[pl.ds(i * 16, 16)] = z16

        def _zden(j, _):
            cid = j * NS + s
            @pl.when(cid < ND)
            def _():
                pltpu.sync_copy(zden_v, den_sp.at[pl.ds(cid * DCH, DCH)])
            return 0
        lax.fori_loop(0, -(-ND // NS), _zden, 0)

    plsc.subcore_barrier()

    def _chunk(j, _):
        cid = j * NS + s

        @pl.when(cid < N_CHUNKS)
        def _():
            pltpu.sync_copy(src_hbm.at[pl.ds(cid * KSUB, KSUB)], src2_v)
            pltpu.sync_copy(dst_hbm.at[pl.ds(cid * KSUB, KSUB)], dst2_v)

            for k in range(KSUB):
                pltpu.sync_copy(as_sp.at[src2_v.at[k]], asv_v.at[k])
                pltpu.sync_copy(ad_sp.at[dst2_v.at[k]], adv_v.at[k])

            for k in range(KSUB):
                for i in range(8):
                    e = (asv_v[k, pl.ds(i * 16, 16)]
                         + adv_v[k, pl.ds(i * 16, 16)])
                    e = jnp.where(e >= 0.0, e, e * jnp.float32(0.2))
                    w_v[k, pl.ds(i * 16, 16)] = jnp.exp(e)

            @pl.when(c == 0)
            def _():
                descs = [
                    pltpu.async_copy(hlo_hbm.at[src2_v.at[k]],
                                     rows_v.at[pl.ds(k * 128, 128)], sem)
                    for k in range(KSUB)
                ]
                for d in descs:
                    d.wait()

            @pl.when(c == 1)
            def _():
                descs = [
                    pltpu.async_copy(hhi_hbm.at[src2_v.at[k]],
                                     rows_v.at[pl.ds(k * 128, 128)], sem)
                    for k in range(KSUB)
                ]
                for d in descs:
                    d.wait()

            for k in range(KSUB):
                kidx = jnp.full((16,), k, jnp.int32)

                def _scale(r2, _):
                    wb = plsc.load_gather(
                        w_v, [kidx, jnp.broadcast_to(r2, (16,))])
                    ri = k * 128 + r2
                    rows_v[ri, pl.ds(0, 16)] = rows_v[ri, pl.ds(0, 16)] * wb
                    rows_v[ri, pl.ds(16, 16)] = rows_v[ri, pl.ds(16, 16)] * wb
                    return 0
                lax.fori_loop(0, 128, _scale, 0)

            for k in range(KSUB):
                pltpu.sync_copy(rows_v.at[pl.ds(k * 128, 128)],
                                acc_sp.at[dst2_v.at[k]], add=True)

            @pl.when(c == 0)
            def _():
                for k in range(KSUB):
                    pltpu.sync_copy(w_v.at[k], den_sp.at[dst2_v.at[k]],
                                    add=True)
        return 0

    lax.fori_loop(0, CHUNK_ITERS, _chunk, 0)
    plsc.subcore_barrier()

    coff = c * N

    def _wacc(j, _):
        cid = j * NS + s
        @pl.when(cid < NZ)
        def _():
            pltpu.sync_copy(acc_sp.at[pl.ds(cid * ZCH, ZCH)],
                            agg_hbm.at[pl.ds(coff + cid * ZCH, ZCH)])
        return 0
    lax.fori_loop(0, -(-NZ // NS), _wacc, 0)

    @pl.when(c == 0)
    def _():
        def _wden(j, _):
            cid = j * NS + s
            @pl.when(cid < ND)
            def _():
                pltpu.sync_copy(den_sp.at[pl.ds(cid * DCH, DCH)],
                                den_hbm.at[pl.ds(cid * DCH, DCH)])
            return 0
        lax.fori_loop(0, -(-ND // NS), _wden, 0)


_sc_edge = functools.partial(
    pl.kernel,
    out_type=[
        jax.ShapeDtypeStruct((2 * N, 32), jnp.float32),
        jax.ShapeDtypeStruct((N2,), jnp.float32),
    ],
    mesh=_MESH,
    scratch_types=[
        pltpu.VMEM((KSUB, 128), jnp.int32),
        pltpu.VMEM((KSUB, 128), jnp.int32),
        pltpu.VMEM((KSUB, 128), jnp.float32),
        pltpu.VMEM((KSUB, 128), jnp.float32),
        pltpu.VMEM((KSUB, 128), jnp.float32),
        pltpu.VMEM((C_EDGE, 32), jnp.float32),
        pltpu.VMEM((DCH,), jnp.float32),
        pltpu.VMEM_SHARED((N,), jnp.float32),
        pltpu.VMEM_SHARED((N,), jnp.float32),
        pltpu.VMEM_SHARED((N, 32), jnp.float32),
        pltpu.VMEM_SHARED((N2,), jnp.float32),
        pltpu.SemaphoreType.DMA,
    ],
    compiler_params=pltpu.CompilerParams(needs_layout_passes=False, use_tc_tiling_on_sc=False),
)(_edge_body)


def _dot_body(x2_hbm, user_hbm, pos_hbm, out_hbm,
              uv, pv, ru, rp, ov, sem):
    c = lax.axis_index("c")
    s = lax.axis_index("s")
    wid = c * NS + s

    pltpu.sync_copy(user_hbm.at[pl.ds(wid * 128, 128)], uv)
    pltpu.sync_copy(pos_hbm.at[pl.ds(wid * 128, 128)], pv)
    for i in range(8):
        pv[pl.ds(i * 16, 16)] = pv[pl.ds(i * 16, 16)] + N_USERS

    pltpu.async_copy(x2_hbm.at[uv], ru, sem).wait()
    pltpu.async_copy(x2_hbm.at[pv], rp, sem).wait()

    lane0 = lax.iota(jnp.int32, 16) == 0

    def _pair(r, _):
        p = ru[r, pl.ds(0, 16)] * rp[r, pl.ds(0, 16)]
        p = p + ru[r, pl.ds(16, 16)] * rp[r, pl.ds(16, 16)]
        p = p + ru[r, pl.ds(32, 16)] * rp[r, pl.ds(32, 16)]
        p = p + ru[r, pl.ds(48, 16)] * rp[r, pl.ds(48, 16)]
        acc = jnp.sum(p, axis=0)
        plsc.store_scatter(ov, [jnp.broadcast_to(r, (16,))],
                           jnp.broadcast_to(acc, (16,)), mask=lane0)
        return 0

    lax.fori_loop(0, 128, _pair, 0)
    pltpu.sync_copy(ov, out_hbm.at[pl.ds(wid * 128, 128)])


_sc_dot = functools.partial(
    pl.kernel,
    out_type=jax.ShapeDtypeStruct((BATCH,), jnp.float32),
    mesh=_MESH,
    scratch_types=[
        pltpu.VMEM((128,), jnp.int32),
        pltpu.VMEM((128,), jnp.int32),
        pltpu.VMEM((128, K), jnp.float32),
        pltpu.VMEM((128, K), jnp.float32),
        pltpu.VMEM((128,), jnp.float32),
        pltpu.SemaphoreType.DMA,
    ],
    compiler_params=pltpu.CompilerParams(needs_layout_passes=False, use_tc_tiling_on_sc=False),
)(_dot_body)


def _layer_glue(al):
    return al[:, 0], al[:, 1]


@jax.jit
def kernel(Gu, Gi, W0, a_src0, a_dst0, b0, W1, a_src1, a_dst1, b1,
           edge_index, user, pos):
    x0 = jnp.concatenate([Gu, Gi], axis=0)
    src2d = edge_index[0].astype(jnp.int32).reshape(E // 128, 128)
    dst2d = edge_index[1].astype(jnp.int32).reshape(E // 128, 128)
    user1 = user.astype(jnp.int32)
    pos1 = pos.astype(jnp.int32)

    def a2pad(a_src, a_dst, W):
        z = jnp.zeros((K, 62), jnp.float32)
        return jnp.concatenate(
            [(W @ a_src.reshape(K))[:, None],
             (W @ a_dst.reshape(K))[:, None], z], axis=1)

    def wsplit(W):
        return W.reshape(K, 2, 32).transpose(1, 0, 2)

    hlo, hhi, al = _tc_prep0(x0, wsplit(W0), a2pad(a_src0, a_dst0, W0))
    asv, adv = _layer_glue(al)
    agg, den = _sc_edge(asv, adv, src2d, dst2d, hlo, hhi)
    aggcat = jnp.concatenate([agg[:N], agg[N:]], axis=1)

    hlo, hhi, al = _tc_prep1(aggcat, den[:N].reshape(N, 1), b0.reshape(1, K),
                             wsplit(W1), a2pad(a_src1, a_dst1, W1))
    asv, adv = _layer_glue(al)
    agg, den = _sc_edge(asv, adv, src2d, dst2d, hlo, hhi)
    aggcat = jnp.concatenate([agg[:N], agg[N:]], axis=1)

    x2 = _tc_final(aggcat, den[:N].reshape(N, 1), b1.reshape(1, K))
    return _sc_dot(x2, user1, pos1)


# speedup vs baseline: 35.8357x; 1.6521x over previous
import functools

import jax
import jax.numpy as jnp
from jax import lax
from jax.experimental import pallas as pl
from jax.experimental.pallas import tpu as pltpu
from jax.experimental.pallas import tpu_sc as plsc

N_USERS = 25000
N_ITEMS = 25000
N = N_USERS + N_ITEMS
E = 800000
K = 64
BATCH = 4096

NS = 16
NC = 2
C_EDGE = 256
KSUB = C_EDGE // 128
N_CHUNKS = E // C_EDGE
CHUNK_ITERS = -(-N_CHUNKS // NS)
ZCH = 200
NZ = N // ZCH
N2 = 50048
DCH = 2176
ND = N2 // DCH

BLK = 5000
GRID_I = N // BLK


def _prep0_body(x_ref, w_ref, a2_ref, hlo_ref, hhi_ref, al_ref):
    x = x_ref[...]
    hlo_ref[...] = jnp.dot(x, w_ref[0], preferred_element_type=jnp.float32)
    hhi_ref[...] = jnp.dot(x, w_ref[1], preferred_element_type=jnp.float32)
    al_ref[...] = jnp.dot(x, a2_ref[...], preferred_element_type=jnp.float32)


_PREP_OUT_SPECS = None


def _prep_specs():
    in_specs = [
        pl.BlockSpec((BLK, K), lambda i: (i, 0)),
        pl.BlockSpec((2, K, 32), lambda i: (0, 0, 0)),
        pl.BlockSpec((K, 64), lambda i: (0, 0)),
    ]
    out_specs = [
        pl.BlockSpec((BLK, 32), lambda i: (i, 0)),
        pl.BlockSpec((BLK, 32), lambda i: (i, 0)),
        pl.BlockSpec((BLK, 64), lambda i: (i, 0)),
    ]
    out_shape = [
        jax.ShapeDtypeStruct((N, 32), jnp.float32),
        jax.ShapeDtypeStruct((N, 32), jnp.float32),
        jax.ShapeDtypeStruct((N, 64), jnp.float32),
    ]
    return in_specs, out_specs, out_shape


def _tc_prep0(x, W2, A2p):
    in_specs, out_specs, out_shape = _prep_specs()
    return pl.pallas_call(
        _prep0_body, grid=(GRID_I,), in_specs=in_specs,
        out_specs=out_specs, out_shape=out_shape,
    )(x, W2, A2p)


def _prep1_body(agg_ref, den_ref, b_ref, w_ref, a2_ref,
                hlo_ref, hhi_ref, al_ref):
    d = den_ref[...]
    x = agg_ref[...] / (d + 1e-16) + b_ref[...]
    x = jnp.where(x > 0, x, jnp.exp(x) - 1.0)
    hlo_ref[...] = jnp.dot(x, w_ref[0], preferred_element_type=jnp.float32)
    hhi_ref[...] = jnp.dot(x, w_ref[1], preferred_element_type=jnp.float32)
    al_ref[...] = jnp.dot(x, a2_ref[...], preferred_element_type=jnp.float32)


def _tc_prep1(agg, den2d, brow, W2, A2p):
    in_specs, out_specs, out_shape = _prep_specs()
    in_specs = [
        in_specs[0],
        pl.BlockSpec((BLK, 1), lambda i: (i, 0)),
        pl.BlockSpec((1, K), lambda i: (0, 0)),
        in_specs[1],
        in_specs[2],
    ]
    return pl.pallas_call(
        _prep1_body, grid=(GRID_I,), in_specs=in_specs,
        out_specs=out_specs, out_shape=out_shape,
    )(agg, den2d, brow, W2, A2p)


def _final_body(agg_ref, den_ref, b_ref, out_ref):
    d = den_ref[...]
    out_ref[...] = agg_ref[...] / (d + 1e-16) + b_ref[...]


def _tc_final(agg, den2d, brow):
    return pl.pallas_call(
        _final_body,
        grid=(GRID_I,),
        in_specs=[
            pl.BlockSpec((BLK, K), lambda i: (i, 0)),
            pl.BlockSpec((BLK, 1), lambda i: (i, 0)),
            pl.BlockSpec((1, K), lambda i: (0, 0)),
        ],
        out_specs=pl.BlockSpec((BLK, K), lambda i: (i, 0)),
        out_shape=jax.ShapeDtypeStruct((N, K), jnp.float32),
    )(agg, den2d, brow)


_MESH = plsc.VectorSubcoreMesh(core_axis_name="c", subcore_axis_name="s")


def _edge_body(as_hbm, ad_hbm, src_hbm, dst_hbm, hlo_hbm, hhi_hbm,
               agg_hbm, den_hbm,
               src2_v, dst2a_v, dst2b_v, asv_v, adv_v, wa_v, wb_v, rows_v,
               zden_v, as_sp, ad_sp, acc_sp, den_sp,
               ga0, ga1, gb0, gb1, sa0, sa1, sb0, sb1, da, db, asem):
    c = lax.axis_index("c")
    s = lax.axis_index("s")

    z16 = jnp.zeros((16,), jnp.float32)

    def _zrow(r, _):
        rows_v[r, pl.ds(0, 16)] = z16
        rows_v[r, pl.ds(16, 16)] = z16
        return 0
    lax.fori_loop(0, 2 * C_EDGE, _zrow, 0)

    def _zacc(j, _):
        cid = j * NS + s
        @pl.when(cid < NZ)
        def _():
            pltpu.sync_copy(rows_v.at[pl.ds(0, ZCH)],
                            acc_sp.at[pl.ds(cid * ZCH, ZCH)])
        return 0
    lax.fori_loop(0, -(-NZ // NS), _zacc, 0)

    @pl.when(s == 0)
    def _():
        pltpu.sync_copy(as_hbm, as_sp)

    @pl.when(s == 1)
    def _():
        pltpu.sync_copy(ad_hbm, ad_sp)

    @pl.when(c == 0)
    def _():
        for i in range(DCH // 16):
            zden_v[pl.ds(i * 16, 16)] = z16

        def _zden(j, _):
            cid = j * NS + s
            @pl.when(cid < ND)
            def _():
                pltpu.sync_copy(zden_v, den_sp.at[pl.ds(cid * DCH, DCH)])
            return 0
        lax.fori_loop(0, -(-ND // NS), _zden, 0)

    plsc.subcore_barrier()

    def _process(jj, cid, roff, dst2_v, w_v, gsems, ssems, dsem):
        @pl.when(cid < N_CHUNKS)
        def _():
            @pl.when(jj > 0)
            def _():
                for g in range(KSUB):
                    pltpu.make_async_copy(
                        hlo_hbm.at[pl.ds(0, 128)],
                        rows_v.at[pl.ds(roff + g * 128, 128)],
                        ssems[g]).wait()

                @pl.when(c == 0)
                def _():
                    for g in range(KSUB):
                        pltpu.make_async_copy(
                            as_hbm.at[pl.ds(0, 128)], w_v.at[g], dsem).wait()

            pltpu.sync_copy(src_hbm.at[pl.ds(cid * KSUB, KSUB)], src2_v)
            pltpu.sync_copy(dst_hbm.at[pl.ds(cid * KSUB, KSUB)], dst2_v)

            @pl.when(c == 0)
            def _():
                for g in range(KSUB):
                    pltpu.async_copy(hlo_hbm.at[src2_v.at[g]],
                                     rows_v.at[pl.ds(roff + g * 128, 128)],
                                     gsems[g])

            @pl.when(c == 1)
            def _():
                for g in range(KSUB):
                    pltpu.async_copy(hhi_hbm.at[src2_v.at[g]],
                                     rows_v.at[pl.ds(roff + g * 128, 128)],
                                     gsems[g])

            adescs = []
            for g in range(KSUB):
                adescs.append(
                    pltpu.async_copy(as_sp.at[src2_v.at[g]], asv_v.at[g],
                                     asem))
                adescs.append(
                    pltpu.async_copy(ad_sp.at[dst2_v.at[g]], adv_v.at[g],
                                     asem))
            for d in adescs:
                d.wait()

            for g in range(KSUB):
                for i in range(8):
                    e = (asv_v[g, pl.ds(i * 16, 16)]
                         + adv_v[g, pl.ds(i * 16, 16)])
                    e = jnp.where(e >= 0.0, e, e * jnp.float32(0.2))
                    w_v[g, pl.ds(i * 16, 16)] = jnp.exp(e)

            for g in range(KSUB):
                pltpu.make_async_copy(
                    hlo_hbm.at[pl.ds(0, 128)],
                    rows_v.at[pl.ds(roff + g * 128, 128)], gsems[g]).wait()
                kidx = jnp.full((16,), g, jnp.int32)

                @plsc.parallel_loop(0, 128, unroll=8)
                def _scale(r2):
                    wb = plsc.load_gather(
                        w_v, [kidx, jnp.broadcast_to(r2, (16,))])
                    ri = roff + g * 128 + r2
                    rows_v[ri, pl.ds(0, 16)] = rows_v[ri, pl.ds(0, 16)] * wb
                    rows_v[ri, pl.ds(16, 16)] = rows_v[ri, pl.ds(16, 16)] * wb

                pltpu.async_copy(rows_v.at[pl.ds(roff + g * 128, 128)],
                                 acc_sp.at[dst2_v.at[g]], ssems[g], add=True)

                @pl.when(c == 0)
                def _():
                    pltpu.async_copy(w_v.at[g], den_sp.at[dst2_v.at[g]],
                                     dsem, add=True)

    def _pair(jj, _):
        _process(jj, (2 * jj) * NS + s, 0, dst2a_v, wa_v,
                 (ga0, ga1), (sa0, sa1), da)
        _process(jj, (2 * jj + 1) * NS + s, C_EDGE, dst2b_v, wb_v,
                 (gb0, gb1), (sb0, sb1), db)
        return 0

    lax.fori_loop(0, -(-CHUNK_ITERS // 2), _pair, 0)

    for roff, ssems in ((0, (sa0, sa1)), (C_EDGE, (sb0, sb1))):
        for g in range(KSUB):
            pltpu.make_async_copy(
                hlo_hbm.at[pl.ds(0, 128)],
                rows_v.at[pl.ds(roff + g * 128, 128)], ssems[g]).wait()

    @pl.when(c == 0)
    def _():
        for w_v, dsem in ((wa_v, da), (wb_v, db)):
            for g in range(KSUB):
                pltpu.make_async_copy(
                    as_hbm.at[pl.ds(0, 128)], w_v.at[g], dsem).wait()

    plsc.subcore_barrier()

    coff = c * N

    def _wacc(j, _):
        cid = j * NS + s
        @pl.when(cid < NZ)
        def _():
            pltpu.sync_copy(acc_sp.at[pl.ds(cid * ZCH, ZCH)],
                            agg_hbm.at[pl.ds(coff + cid * ZCH, ZCH)])
        return 0
    lax.fori_loop(0, -(-NZ // NS), _wacc, 0)

    @pl.when(c == 0)
    def _():
        def _wden(j, _):
            cid = j * NS + s
            @pl.when(cid < ND)
            def _():
                pltpu.sync_copy(den_sp.at[pl.ds(cid * DCH, DCH)],
                                den_hbm.at[pl.ds(cid * DCH, DCH)])
            return 0
        lax.fori_loop(0, -(-ND // NS), _wden, 0)


_sc_edge = functools.partial(
    pl.kernel,
    out_type=[
        jax.ShapeDtypeStruct((2 * N, 32), jnp.float32),
        jax.ShapeDtypeStruct((N2,), jnp.float32),
    ],
    mesh=_MESH,
    scratch_types=[
        pltpu.VMEM((KSUB, 128), jnp.int32),
        pltpu.VMEM((KSUB, 128), jnp.int32),
        pltpu.VMEM((KSUB, 128), jnp.int32),
        pltpu.VMEM((KSUB, 128), jnp.float32),
        pltpu.VMEM((KSUB, 128), jnp.float32),
        pltpu.VMEM((KSUB, 128), jnp.float32),
        pltpu.VMEM((KSUB, 128), jnp.float32),
        pltpu.VMEM((2 * C_EDGE, 32), jnp.float32),
        pltpu.VMEM((DCH,), jnp.float32),
        pltpu.VMEM_SHARED((N,), jnp.float32),
        pltpu.VMEM_SHARED((N,), jnp.float32),
        pltpu.VMEM_SHARED((N, 32), jnp.float32),
        pltpu.VMEM_SHARED((N2,), jnp.float32),
    ] + [pltpu.SemaphoreType.DMA] * 11,
    compiler_params=pltpu.CompilerParams(needs_layout_passes=False, use_tc_tiling_on_sc=False),
)(_edge_body)


def _dot_body(x2_hbm, user_hbm, pos_hbm, out_hbm,
              uv, pv, ru, rp, ov, sem):
    c = lax.axis_index("c")
    s = lax.axis_index("s")
    wid = c * NS + s

    pltpu.sync_copy(user_hbm.at[pl.ds(wid * 128, 128)], uv)
    pltpu.sync_copy(pos_hbm.at[pl.ds(wid * 128, 128)], pv)
    for i in range(8):
        pv[pl.ds(i * 16, 16)] = pv[pl.ds(i * 16, 16)] + N_USERS

    pltpu.async_copy(x2_hbm.at[uv], ru, sem).wait()
    pltpu.async_copy(x2_hbm.at[pv], rp, sem).wait()

    lane0 = lax.iota(jnp.int32, 16) == 0

    def _pair(r, _):
        p = ru[r, pl.ds(0, 16)] * rp[r, pl.ds(0, 16)]
        p = p + ru[r, pl.ds(16, 16)] * rp[r, pl.ds(16, 16)]
        p = p + ru[r, pl.ds(32, 16)] * rp[r, pl.ds(32, 16)]
        p = p + ru[r, pl.ds(48, 16)] * rp[r, pl.ds(48, 16)]
        acc = jnp.sum(p, axis=0)
        plsc.store_scatter(ov, [jnp.broadcast_to(r, (16,))],
                           jnp.broadcast_to(acc, (16,)), mask=lane0)
        return 0

    lax.fori_loop(0, 128, _pair, 0)
    pltpu.sync_copy(ov, out_hbm.at[pl.ds(wid * 128, 128)])


_sc_dot = functools.partial(
    pl.kernel,
    out_type=jax.ShapeDtypeStruct((BATCH,), jnp.float32),
    mesh=_MESH,
    scratch_types=[
        pltpu.VMEM((128,), jnp.int32),
        pltpu.VMEM((128,), jnp.int32),
        pltpu.VMEM((128, K), jnp.float32),
        pltpu.VMEM((128, K), jnp.float32),
        pltpu.VMEM((128,), jnp.float32),
        pltpu.SemaphoreType.DMA,
    ],
    compiler_params=pltpu.CompilerParams(needs_layout_passes=False, use_tc_tiling_on_sc=False),
)(_dot_body)


def _layer_glue(al):
    return al[:, 0], al[:, 1]


@jax.jit
def kernel(Gu, Gi, W0, a_src0, a_dst0, b0, W1, a_src1, a_dst1, b1,
           edge_index, user, pos):
    x0 = jnp.concatenate([Gu, Gi], axis=0)
    src2d = edge_index[0].astype(jnp.int32).reshape(E // 128, 128)
    dst2d = edge_index[1].astype(jnp.int32).reshape(E // 128, 128)
    user1 = user.astype(jnp.int32)
    pos1 = pos.astype(jnp.int32)

    def a2pad(a_src, a_dst, W):
        z = jnp.zeros((K, 62), jnp.float32)
        return jnp.concatenate(
            [(W @ a_src.reshape(K))[:, None],
             (W @ a_dst.reshape(K))[:, None], z], axis=1)

    def wsplit(W):
        return W.reshape(K, 2, 32).transpose(1, 0, 2)

    hlo, hhi, al = _tc_prep0(x0, wsplit(W0), a2pad(a_src0, a_dst0, W0))
    asv, adv = _layer_glue(al)
    agg, den = _sc_edge(asv, adv, src2d, dst2d, hlo, hhi)
    aggcat = jnp.concatenate([agg[:N], agg[N:]], axis=1)

    hlo, hhi, al = _tc_prep1(aggcat, den[:N].reshape(N, 1), b0.reshape(1, K),
                             wsplit(W1), a2pad(a_src1, a_dst1, W1))
    asv, adv = _layer_glue(al)
    agg, den = _sc_edge(asv, adv, src2d, dst2d, hlo, hhi)
    aggcat = jnp.concatenate([agg[:N], agg[N:]], axis=1)

    x2 = _tc_final(aggcat, den[:N].reshape(N, 1), b1.reshape(1, K))
    return _sc_dot(x2, user1, pos1)


# speedup vs baseline: 41.4395x; 1.1564x over previous
import functools

import jax
import jax.numpy as jnp
from jax import lax
from jax.experimental import pallas as pl
from jax.experimental.pallas import tpu as pltpu
from jax.experimental.pallas import tpu_sc as plsc

N_USERS = 25000
N_ITEMS = 25000
N = N_USERS + N_ITEMS
E = 800000
K = 64
BATCH = 4096

NS = 16
NC = 2
C_EDGE = 256
KSUB = C_EDGE // 128
N_CHUNKS = E // C_EDGE
CHUNK_ITERS = -(-N_CHUNKS // NS)
ZCH = 200
NZ = N // ZCH
N2 = 50048
DCH = 2176
ND = N2 // DCH

BLK = 5000
GRID_I = N // BLK


def _prep0_body(x_ref, w_ref, a2_ref, hlo_ref, hhi_ref, al_ref):
    x = x_ref[...]
    hlo_ref[...] = jnp.dot(x, w_ref[0], preferred_element_type=jnp.float32)
    hhi_ref[...] = jnp.dot(x, w_ref[1], preferred_element_type=jnp.float32)
    al_ref[...] = jnp.dot(x, a2_ref[...], preferred_element_type=jnp.float32)


_PREP_OUT_SPECS = None


def _prep_specs():
    in_specs = [
        pl.BlockSpec((BLK, K), lambda i: (i, 0)),
        pl.BlockSpec((2, K, 32), lambda i: (0, 0, 0)),
        pl.BlockSpec((K, 64), lambda i: (0, 0)),
    ]
    out_specs = [
        pl.BlockSpec((BLK, 32), lambda i: (i, 0)),
        pl.BlockSpec((BLK, 32), lambda i: (i, 0)),
        pl.BlockSpec((BLK, 64), lambda i: (i, 0)),
    ]
    out_shape = [
        jax.ShapeDtypeStruct((N, 32), jnp.float32),
        jax.ShapeDtypeStruct((N, 32), jnp.float32),
        jax.ShapeDtypeStruct((N, 64), jnp.float32),
    ]
    return in_specs, out_specs, out_shape


def _tc_prep0(x, W2, A2p):
    in_specs, out_specs, out_shape = _prep_specs()
    return pl.pallas_call(
        _prep0_body, grid=(GRID_I,), in_specs=in_specs,
        out_specs=out_specs, out_shape=out_shape,
    )(x, W2, A2p)


def _prep1_body(aglo_ref, aghi_ref, den_ref, b_ref, w_ref, a2_ref,
                hlo_ref, hhi_ref, al_ref):
    d = den_ref[...]
    agg = jnp.concatenate([aglo_ref[...], aghi_ref[...]], axis=1)
    x = agg / (d + 1e-16) + b_ref[...]
    x = jnp.where(x > 0, x, jnp.exp(x) - 1.0)
    hlo_ref[...] = jnp.dot(x, w_ref[0], preferred_element_type=jnp.float32)
    hhi_ref[...] = jnp.dot(x, w_ref[1], preferred_element_type=jnp.float32)
    al_ref[...] = jnp.dot(x, a2_ref[...], preferred_element_type=jnp.float32)


def _tc_prep1(agg2n, den2d, brow, W2, A2p):
    in_specs, out_specs, out_shape = _prep_specs()
    in_specs = [
        pl.BlockSpec((BLK, 32), lambda i: (i, 0)),
        pl.BlockSpec((BLK, 32), lambda i: (i + GRID_I, 0)),
        pl.BlockSpec((BLK, 1), lambda i: (i, 0)),
        pl.BlockSpec((1, K), lambda i: (0, 0)),
        in_specs[1],
        in_specs[2],
    ]
    return pl.pallas_call(
        _prep1_body, grid=(GRID_I,), in_specs=in_specs,
        out_specs=out_specs, out_shape=out_shape,
    )(agg2n, agg2n, den2d, brow, W2, A2p)


_MESH = plsc.VectorSubcoreMesh(core_axis_name="c", subcore_axis_name="s")


def _edge_body(as_hbm, ad_hbm, src_hbm, dst_hbm, hlo_hbm, hhi_hbm,
               agg_hbm, den_hbm,
               src2_v, dst2a_v, dst2b_v, asv_v, adv_v, wa_v, wb_v, rows_v,
               zden_v, as_sp, ad_sp, acc_sp, den_sp,
               ga0, ga1, gb0, gb1, sa0, sa1, sb0, sb1, da, db, asem):
    c = lax.axis_index("c")
    s = lax.axis_index("s")

    z16 = jnp.zeros((16,), jnp.float32)

    def _zrow(r, _):
        rows_v[r, pl.ds(0, 16)] = z16
        rows_v[r, pl.ds(16, 16)] = z16
        return 0
    lax.fori_loop(0, 2 * C_EDGE, _zrow, 0)

    def _zacc(j, _):
        cid = j * NS + s
        @pl.when(cid < NZ)
        def _():
            pltpu.sync_copy(rows_v.at[pl.ds(0, ZCH)],
                            acc_sp.at[pl.ds(cid * ZCH, ZCH)])
        return 0
    lax.fori_loop(0, -(-NZ // NS), _zacc, 0)

    @pl.when(s == 0)
    def _():
        pltpu.sync_copy(as_hbm, as_sp)

    @pl.when(s == 1)
    def _():
        pltpu.sync_copy(ad_hbm, ad_sp)

    @pl.when(c == 0)
    def _():
        for i in range(DCH // 16):
            zden_v[pl.ds(i * 16, 16)] = z16

        def _zden(j, _):
            cid = j * NS + s
            @pl.when(cid < ND)
            def _():
                pltpu.sync_copy(zden_v, den_sp.at[pl.ds(cid * DCH, DCH)])
            return 0
        lax.fori_loop(0, -(-ND // NS), _zden, 0)

    plsc.subcore_barrier()

    def _process(jj, cid, roff, dst2_v, w_v, gsems, ssems, dsem):
        @pl.when(cid < N_CHUNKS)
        def _():
            @pl.when(jj > 0)
            def _():
                for g in range(KSUB):
                    pltpu.make_async_copy(
                        hlo_hbm.at[pl.ds(0, 128)],
                        rows_v.at[pl.ds(roff + g * 128, 128)],
                        ssems[g]).wait()

                @pl.when(c == 0)
                def _():
                    for g in range(KSUB):
                        pltpu.make_async_copy(
                            as_hbm.at[pl.ds(0, 128)], w_v.at[g], dsem).wait()

            pltpu.sync_copy(src_hbm.at[pl.ds(cid * KSUB, KSUB)], src2_v)
            pltpu.sync_copy(dst_hbm.at[pl.ds(cid * KSUB, KSUB)], dst2_v)

            @pl.when(c == 0)
            def _():
                for g in range(KSUB):
                    pltpu.async_copy(hlo_hbm.at[src2_v.at[g]],
                                     rows_v.at[pl.ds(roff + g * 128, 128)],
                                     gsems[g])

            @pl.when(c == 1)
            def _():
                for g in range(KSUB):
                    pltpu.async_copy(hhi_hbm.at[src2_v.at[g]],
                                     rows_v.at[pl.ds(roff + g * 128, 128)],
                                     gsems[g])

            adescs = []
            for g in range(KSUB):
                adescs.append(
                    pltpu.async_copy(as_sp.at[src2_v.at[g]], asv_v.at[g],
                                     asem))
                adescs.append(
                    pltpu.async_copy(ad_sp.at[dst2_v.at[g]], adv_v.at[g],
                                     asem))
            for d in adescs:
                d.wait()

            for g in range(KSUB):
                for i in range(8):
                    e = (asv_v[g, pl.ds(i * 16, 16)]
                         + adv_v[g, pl.ds(i * 16, 16)])
                    e = jnp.where(e >= 0.0, e, e * jnp.float32(0.2))
                    w_v[g, pl.ds(i * 16, 16)] = jnp.exp(e)

            for g in range(KSUB):
                pltpu.make_async_copy(
                    hlo_hbm.at[pl.ds(0, 128)],
                    rows_v.at[pl.ds(roff + g * 128, 128)], gsems[g]).wait()
                kidx = jnp.full((16,), g, jnp.int32)

                @plsc.parallel_loop(0, 128, unroll=8)
                def _scale(r2):
                    wb = plsc.load_gather(
                        w_v, [kidx, jnp.broadcast_to(r2, (16,))])
                    ri = roff + g * 128 + r2
                    rows_v[ri, pl.ds(0, 16)] = rows_v[ri, pl.ds(0, 16)] * wb
                    rows_v[ri, pl.ds(16, 16)] = rows_v[ri, pl.ds(16, 16)] * wb

                pltpu.async_copy(rows_v.at[pl.ds(roff + g * 128, 128)],
                                 acc_sp.at[dst2_v.at[g]], ssems[g], add=True)

                @pl.when(c == 0)
                def _():
                    pltpu.async_copy(w_v.at[g], den_sp.at[dst2_v.at[g]],
                                     dsem, add=True)

    def _pair(jj, _):
        _process(jj, (2 * jj) * NS + s, 0, dst2a_v, wa_v,
                 (ga0, ga1), (sa0, sa1), da)
        _process(jj, (2 * jj + 1) * NS + s, C_EDGE, dst2b_v, wb_v,
                 (gb0, gb1), (sb0, sb1), db)
        return 0

    lax.fori_loop(0, -(-CHUNK_ITERS // 2), _pair, 0)

    for roff, ssems in ((0, (sa0, sa1)), (C_EDGE, (sb0, sb1))):
        for g in range(KSUB):
            pltpu.make_async_copy(
                hlo_hbm.at[pl.ds(0, 128)],
                rows_v.at[pl.ds(roff + g * 128, 128)], ssems[g]).wait()

    @pl.when(c == 0)
    def _():
        for w_v, dsem in ((wa_v, da), (wb_v, db)):
            for g in range(KSUB):
                pltpu.make_async_copy(
                    as_hbm.at[pl.ds(0, 128)], w_v.at[g], dsem).wait()

    plsc.subcore_barrier()

    coff = c * N

    def _wacc(j, _):
        cid = j * NS + s
        @pl.when(cid < NZ)
        def _():
            pltpu.sync_copy(acc_sp.at[pl.ds(cid * ZCH, ZCH)],
                            agg_hbm.at[pl.ds(coff + cid * ZCH, ZCH)])
        return 0
    lax.fori_loop(0, -(-NZ // NS), _wacc, 0)

    @pl.when(c == 0)
    def _():
        def _wden(j, _):
            cid = j * NS + s
            @pl.when(cid < ND)
            def _():
                pltpu.sync_copy(den_sp.at[pl.ds(cid * DCH, DCH)],
                                den_hbm.at[pl.ds(cid * DCH, DCH)])
            return 0
        lax.fori_loop(0, -(-ND // NS), _wden, 0)


_sc_edge = functools.partial(
    pl.kernel,
    out_type=[
        jax.ShapeDtypeStruct((2 * N, 32), jnp.float32),
        jax.ShapeDtypeStruct((N2,), jnp.float32),
    ],
    mesh=_MESH,
    scratch_types=[
        pltpu.VMEM((KSUB, 128), jnp.int32),
        pltpu.VMEM((KSUB, 128), jnp.int32),
        pltpu.VMEM((KSUB, 128), jnp.int32),
        pltpu.VMEM((KSUB, 128), jnp.float32),
        pltpu.VMEM((KSUB, 128), jnp.float32),
        pltpu.VMEM((KSUB, 128), jnp.float32),
        pltpu.VMEM((KSUB, 128), jnp.float32),
        pltpu.VMEM((2 * C_EDGE, 32), jnp.float32),
        pltpu.VMEM((DCH,), jnp.float32),
        pltpu.VMEM_SHARED((N,), jnp.float32),
        pltpu.VMEM_SHARED((N,), jnp.float32),
        pltpu.VMEM_SHARED((N, 32), jnp.float32),
        pltpu.VMEM_SHARED((N2,), jnp.float32),
    ] + [pltpu.SemaphoreType.DMA] * 11,
    compiler_params=pltpu.CompilerParams(needs_layout_passes=False, use_tc_tiling_on_sc=False),
)(_edge_body)


def _dot_body(agg_hbm, den_hbm, b_hbm, user_hbm, pos_hbm, out_hbm,
              uv, pv, uvh, pvh, rulo, ruhi, rplo, rphi, den_v, b_v, ov, sem):
    c = lax.axis_index("c")
    s = lax.axis_index("s")
    wid = c * NS + s

    pltpu.sync_copy(den_hbm, den_v)
    pltpu.sync_copy(b_hbm, b_v)
    pltpu.sync_copy(user_hbm.at[pl.ds(wid * 128, 128)], uv)
    pltpu.sync_copy(pos_hbm.at[pl.ds(wid * 128, 128)], pv)
    for i in range(8):
        pv[pl.ds(i * 16, 16)] = pv[pl.ds(i * 16, 16)] + N_USERS
        uvh[pl.ds(i * 16, 16)] = uv[pl.ds(i * 16, 16)] + N
    for i in range(8):
        pvh[pl.ds(i * 16, 16)] = pv[pl.ds(i * 16, 16)] + N

    d1 = pltpu.async_copy(agg_hbm.at[uv], rulo, sem)
    d2 = pltpu.async_copy(agg_hbm.at[pv], rplo, sem)
    d3 = pltpu.async_copy(agg_hbm.at[uvh], ruhi, sem)
    d4 = pltpu.async_copy(agg_hbm.at[pvh], rphi, sem)
    for d in (d1, d2, d3, d4):
        d.wait()

    bsegs = [b_v[pl.ds(i * 16, 16)] for i in range(4)]
    eps = jnp.float32(1e-16)

    @plsc.parallel_loop(0, 128, unroll=4)
    def _pair(r):
        u16 = plsc.load_gather(uv, [jnp.broadcast_to(r, (16,))])
        p16 = plsc.load_gather(pv, [jnp.broadcast_to(r, (16,))])
        du = jnp.float32(1.0) / (plsc.load_gather(den_v, [u16]) + eps)
        dp = jnp.float32(1.0) / (plsc.load_gather(den_v, [p16]) + eps)
        acc = jnp.zeros((16,), jnp.float32)
        for seg, ru, rp in ((0, rulo, rplo), (1, rulo, rplo),
                            (2, ruhi, rphi), (3, ruhi, rphi)):
            off = (seg % 2) * 16
            xu = ru[r, pl.ds(off, 16)] * du + bsegs[seg]
            xp = rp[r, pl.ds(off, 16)] * dp + bsegs[seg]
            acc = acc + xu * xp
        a = jnp.sum(acc, axis=0)
        plsc.store_scatter(ov, [jnp.broadcast_to(r, (16,))],
                           jnp.broadcast_to(a, (16,)),
                           mask=lax.iota(jnp.int32, 16) == 0)

    pltpu.sync_copy(ov, out_hbm.at[pl.ds(wid * 128, 128)])


_sc_dot = functools.partial(
    pl.kernel,
    out_type=jax.ShapeDtypeStruct((BATCH,), jnp.float32),
    mesh=_MESH,
    scratch_types=[
        pltpu.VMEM((128,), jnp.int32),
        pltpu.VMEM((128,), jnp.int32),
        pltpu.VMEM((128,), jnp.int32),
        pltpu.VMEM((128,), jnp.int32),
        pltpu.VMEM((128, 32), jnp.float32),
        pltpu.VMEM((128, 32), jnp.float32),
        pltpu.VMEM((128, 32), jnp.float32),
        pltpu.VMEM((128, 32), jnp.float32),
        pltpu.VMEM((N2,), jnp.float32),
        pltpu.VMEM((K,), jnp.float32),
        pltpu.VMEM((128,), jnp.float32),
        pltpu.SemaphoreType.DMA,
    ],
    compiler_params=pltpu.CompilerParams(needs_layout_passes=False, use_tc_tiling_on_sc=False),
)(_dot_body)


def _layer_glue(al):
    return al[:, 0], al[:, 1]


@jax.jit
def kernel(Gu, Gi, W0, a_src0, a_dst0, b0, W1, a_src1, a_dst1, b1,
           edge_index, user, pos):
    x0 = jnp.concatenate([Gu, Gi], axis=0)
    src2d = edge_index[0].astype(jnp.int32).reshape(E // 128, 128)
    dst2d = edge_index[1].astype(jnp.int32).reshape(E // 128, 128)
    user1 = user.astype(jnp.int32)
    pos1 = pos.astype(jnp.int32)

    def a2pad(a_src, a_dst, W):
        z = jnp.zeros((K, 62), jnp.float32)
        return jnp.concatenate(
            [(W @ a_src.reshape(K))[:, None],
             (W @ a_dst.reshape(K))[:, None], z], axis=1)

    def wsplit(W):
        return W.reshape(K, 2, 32).transpose(1, 0, 2)

    hlo, hhi, al = _tc_prep0(x0, wsplit(W0), a2pad(a_src0, a_dst0, W0))
    asv, adv = _layer_glue(al)
    agg, den = _sc_edge(asv, adv, src2d, dst2d, hlo, hhi)

    hlo, hhi, al = _tc_prep1(agg, den[:N].reshape(N, 1), b0.reshape(1, K),
                             wsplit(W1), a2pad(a_src1, a_dst1, W1))
    asv, adv = _layer_glue(al)
    agg, den = _sc_edge(asv, adv, src2d, dst2d, hlo, hhi)

    return _sc_dot(agg, den, b1, user1, pos1)


# speedup vs baseline: 48.6503x; 1.1740x over previous
import functools

import jax
import jax.numpy as jnp
from jax import lax
from jax.experimental import pallas as pl
from jax.experimental.pallas import tpu as pltpu
from jax.experimental.pallas import tpu_sc as plsc

N_USERS = 25000
N_ITEMS = 25000
N = N_USERS + N_ITEMS
E = 800000
K = 64
BATCH = 4096

NS = 16
NC = 2
C_EDGE = 256
KSUB = C_EDGE // 128
N_CHUNKS = E // C_EDGE
CHUNK_ITERS = -(-N_CHUNKS // NS)
ZCH = 200
NZ = N // ZCH
N2 = 50048
DCH = 2176
ND = N2 // DCH

BLK = 5000
GRID_I = N // BLK


def _prep0_body(x_ref, w_ref, a2_ref, hlo_ref, hhi_ref, al_ref):
    x = x_ref[...]
    hlo_ref[...] = jnp.dot(x, w_ref[0], preferred_element_type=jnp.float32)
    hhi_ref[...] = jnp.dot(x, w_ref[1], preferred_element_type=jnp.float32)
    al_ref[...] = jnp.dot(x, a2_ref[...], preferred_element_type=jnp.float32)


_PREP_OUT_SPECS = None


def _prep_specs():
    in_specs = [
        pl.BlockSpec((BLK, K), lambda i: (i, 0)),
        pl.BlockSpec((2, K, 32), lambda i: (0, 0, 0)),
        pl.BlockSpec((K, 64), lambda i: (0, 0)),
    ]
    out_specs = [
        pl.BlockSpec((BLK, 32), lambda i: (i, 0)),
        pl.BlockSpec((BLK, 32), lambda i: (i, 0)),
        pl.BlockSpec((BLK, 64), lambda i: (i, 0)),
    ]
    out_shape = [
        jax.ShapeDtypeStruct((N, 32), jnp.float32),
        jax.ShapeDtypeStruct((N, 32), jnp.float32),
        jax.ShapeDtypeStruct((N, 64), jnp.float32),
    ]
    return in_specs, out_specs, out_shape


def _tc_prep0(x, W2, A2p):
    in_specs, out_specs, out_shape = _prep_specs()
    return pl.pallas_call(
        _prep0_body, grid=(GRID_I,), in_specs=in_specs,
        out_specs=out_specs, out_shape=out_shape,
    )(x, W2, A2p)


def _prep1_body(aglo_ref, aghi_ref, den_ref, b_ref, w_ref, a2_ref,
                hlo_ref, hhi_ref, al_ref):
    d = den_ref[...]
    agg = jnp.concatenate([aglo_ref[...], aghi_ref[...]], axis=1)
    x = agg / (d + 1e-16) + b_ref[...]
    x = jnp.where(x > 0, x, jnp.exp(x) - 1.0)
    hlo_ref[...] = jnp.dot(x, w_ref[0], preferred_element_type=jnp.float32)
    hhi_ref[...] = jnp.dot(x, w_ref[1], preferred_element_type=jnp.float32)
    al_ref[...] = jnp.dot(x, a2_ref[...], preferred_element_type=jnp.float32)


def _tc_prep1(agg2n, den2d, brow, W2, A2p):
    in_specs, out_specs, out_shape = _prep_specs()
    in_specs = [
        pl.BlockSpec((BLK, 32), lambda i: (i, 0)),
        pl.BlockSpec((BLK, 32), lambda i: (i + GRID_I, 0)),
        pl.BlockSpec((BLK, 1), lambda i: (i, 0)),
        pl.BlockSpec((1, K), lambda i: (0, 0)),
        in_specs[1],
        in_specs[2],
    ]
    return pl.pallas_call(
        _prep1_body, grid=(GRID_I,), in_specs=in_specs,
        out_specs=out_specs, out_shape=out_shape,
    )(agg2n, agg2n, den2d, brow, W2, A2p)


_MESH = plsc.VectorSubcoreMesh(core_axis_name="c", subcore_axis_name="s")


def _edge_body(as_hbm, ad_hbm, src_hbm, dst_hbm, hlo_hbm, hhi_hbm,
               agg_hbm, den_hbm,
               src2a_v, src2b_v, dst2a_v, dst2b_v, asva_v, adva_v,
               asvb_v, advb_v, wa_v, wb_v, rows_v,
               zden_v, as_sp, ad_sp, acc_sp, den_sp,
               ga0, ga1, gb0, gb1, sa0, sa1, sb0, sb1, da, db,
               asema, asemb):
    c = lax.axis_index("c")
    s = lax.axis_index("s")

    z16 = jnp.zeros((16,), jnp.float32)

    def _zrow(r, _):
        rows_v[r, pl.ds(0, 16)] = z16
        rows_v[r, pl.ds(16, 16)] = z16
        return 0
    lax.fori_loop(0, 2 * C_EDGE, _zrow, 0)

    def _zacc(j, _):
        cid = j * NS + s
        @pl.when(cid < NZ)
        def _():
            pltpu.sync_copy(rows_v.at[pl.ds(0, ZCH)],
                            acc_sp.at[pl.ds(cid * ZCH, ZCH)])
        return 0
    lax.fori_loop(0, -(-NZ // NS), _zacc, 0)

    @pl.when(s == 0)
    def _():
        pltpu.sync_copy(as_hbm, as_sp)

    @pl.when(s == 1)
    def _():
        pltpu.sync_copy(ad_hbm, ad_sp)

    @pl.when(c == 0)
    def _():
        for i in range(DCH // 16):
            zden_v[pl.ds(i * 16, 16)] = z16

        def _zden(j, _):
            cid = j * NS + s
            @pl.when(cid < ND)
            def _():
                pltpu.sync_copy(zden_v, den_sp.at[pl.ds(cid * DCH, DCH)])
            return 0
        lax.fori_loop(0, -(-ND // NS), _zden, 0)

    plsc.subcore_barrier()

    PAIRS = -(-CHUNK_ITERS // 2)

    def _cid_a(jj):
        return (2 * jj) * NS + s

    def _cid_b(jj):
        return (2 * jj + 1) * NS + s

    def _load_src_as(cid, src2_v, asv_v, asem):
        pltpu.sync_copy(src_hbm.at[pl.ds(cid * KSUB, KSUB)], src2_v)
        for g in range(KSUB):
            pltpu.async_copy(as_sp.at[src2_v.at[g]], asv_v.at[g], asem)

    def _load_dst_ad(cid, dst2_v, adv_v, asem):
        pltpu.sync_copy(dst_hbm.at[pl.ds(cid * KSUB, KSUB)], dst2_v)
        for g in range(KSUB):
            pltpu.async_copy(ad_sp.at[dst2_v.at[g]], adv_v.at[g], asem)

    def _issue_row_gathers(src2_v, roff, gsems):
        @pl.when(c == 0)
        def _():
            for g in range(KSUB):
                pltpu.async_copy(hlo_hbm.at[src2_v.at[g]],
                                 rows_v.at[pl.ds(roff + g * 128, 128)],
                                 gsems[g])

        @pl.when(c == 1)
        def _():
            for g in range(KSUB):
                pltpu.async_copy(hhi_hbm.at[src2_v.at[g]],
                                 rows_v.at[pl.ds(roff + g * 128, 128)],
                                 gsems[g])

    def _drain_scatters(roff, ssems, dsem, w_v):
        for g in range(KSUB):
            pltpu.make_async_copy(
                hlo_hbm.at[pl.ds(0, 128)],
                rows_v.at[pl.ds(roff + g * 128, 128)], ssems[g]).wait()

        @pl.when(c == 0)
        def _():
            for g in range(KSUB):
                pltpu.make_async_copy(
                    as_hbm.at[pl.ds(0, 128)], w_v.at[g], dsem).wait()

    def _compute(asv_v, adv_v, w_v, dst2_v, roff, gsems, ssems, dsem, asem):
        for g in range(KSUB):
            pltpu.make_async_copy(as_hbm.at[pl.ds(0, 128)], asv_v.at[g],
                                  asem).wait()
            pltpu.make_async_copy(as_hbm.at[pl.ds(0, 128)], adv_v.at[g],
                                  asem).wait()

        for g in range(KSUB):
            for i in range(8):
                e = (asv_v[g, pl.ds(i * 16, 16)]
                     + adv_v[g, pl.ds(i * 16, 16)])
                e = jnp.where(e >= 0.0, e, e * jnp.float32(0.2))
                w_v[g, pl.ds(i * 16, 16)] = jnp.exp(e)

        for g in range(KSUB):
            pltpu.make_async_copy(
                hlo_hbm.at[pl.ds(0, 128)],
                rows_v.at[pl.ds(roff + g * 128, 128)], gsems[g]).wait()
            kidx = jnp.full((16,), g, jnp.int32)

            @plsc.parallel_loop(0, 128, unroll=8)
            def _scale(r2):
                wb = plsc.load_gather(
                    w_v, [kidx, jnp.broadcast_to(r2, (16,))])
                ri = roff + g * 128 + r2
                rows_v[ri, pl.ds(0, 16)] = rows_v[ri, pl.ds(0, 16)] * wb
                rows_v[ri, pl.ds(16, 16)] = rows_v[ri, pl.ds(16, 16)] * wb

            pltpu.async_copy(rows_v.at[pl.ds(roff + g * 128, 128)],
                             acc_sp.at[dst2_v.at[g]], ssems[g], add=True)

            @pl.when(c == 0)
            def _():
                pltpu.async_copy(w_v.at[g], den_sp.at[dst2_v.at[g]],
                                 dsem, add=True)

    SEMS_A = (ga0, ga1)
    SEMS_B = (gb0, gb1)
    SSEMS_A = (sa0, sa1)
    SSEMS_B = (sb0, sb1)

    _load_src_as(s, src2a_v, asva_v, asema)
    _load_dst_ad(s, dst2a_v, adva_v, asema)
    _issue_row_gathers(src2a_v, 0, SEMS_A)

    def _pair(jj, _):
        cid_a, cid_b = _cid_a(jj), _cid_b(jj)
        cid_a2 = (2 * jj + 2) * NS + s

        @pl.when(cid_b < N_CHUNKS)
        def _():
            _load_src_as(cid_b, src2b_v, asvb_v, asemb)

        @pl.when(cid_a < N_CHUNKS)
        def _():
            _compute(asva_v, adva_v, wa_v, dst2a_v, 0, SEMS_A, SSEMS_A, da,
                     asema)

        @pl.when((jj > 0) & (cid_b < N_CHUNKS))
        def _():
            _drain_scatters(C_EDGE, SSEMS_B, db, wb_v)

        @pl.when(cid_b < N_CHUNKS)
        def _():
            _load_dst_ad(cid_b, dst2b_v, advb_v, asemb)
            _issue_row_gathers(src2b_v, C_EDGE, SEMS_B)

        @pl.when(cid_a2 < N_CHUNKS)
        def _():
            _load_src_as(cid_a2, src2a_v, asva_v, asema)

        @pl.when(cid_b < N_CHUNKS)
        def _():
            _compute(asvb_v, advb_v, wb_v, dst2b_v, C_EDGE, SEMS_B, SSEMS_B,
                     db, asemb)

        @pl.when(cid_a < N_CHUNKS)
        def _():
            _drain_scatters(0, SSEMS_A, da, wa_v)

        @pl.when(cid_a2 < N_CHUNKS)
        def _():
            _load_dst_ad(cid_a2, dst2a_v, adva_v, asema)
            _issue_row_gathers(src2a_v, 0, SEMS_A)

        return 0

    lax.fori_loop(0, PAIRS, _pair, 0)

    _drain_scatters(C_EDGE, SSEMS_B, db, wb_v)

    plsc.subcore_barrier()

    coff = c * N

    def _wacc(j, _):
        cid = j * NS + s
        @pl.when(cid < NZ)
        def _():
            pltpu.sync_copy(acc_sp.at[pl.ds(cid * ZCH, ZCH)],
                            agg_hbm.at[pl.ds(coff + cid * ZCH, ZCH)])
        return 0
    lax.fori_loop(0, -(-NZ // NS), _wacc, 0)

    @pl.when(c == 0)
    def _():
        def _wden(j, _):
            cid = j * NS + s
            @pl.when(cid < ND)
            def _():
                pltpu.sync_copy(den_sp.at[pl.ds(cid * DCH, DCH)],
                                den_hbm.at[pl.ds(cid * DCH, DCH)])
            return 0
        lax.fori_loop(0, -(-ND // NS), _wden, 0)


_sc_edge = functools.partial(
    pl.kernel,
    out_type=[
        jax.ShapeDtypeStruct((2 * N, 32), jnp.float32),
        jax.ShapeDtypeStruct((N2,), jnp.float32),
    ],
    mesh=_MESH,
    scratch_types=[
        pltpu.VMEM((KSUB, 128), jnp.int32),
        pltpu.VMEM((KSUB, 128), jnp.int32),
        pltpu.VMEM((KSUB, 128), jnp.int32),
        pltpu.VMEM((KSUB, 128), jnp.int32),
        pltpu.VMEM((KSUB, 128), jnp.float32),
        pltpu.VMEM((KSUB, 128), jnp.float32),
        pltpu.VMEM((KSUB, 128), jnp.float32),
        pltpu.VMEM((KSUB, 128), jnp.float32),
        pltpu.VMEM((KSUB, 128), jnp.float32),
        pltpu.VMEM((KSUB, 128), jnp.float32),
        pltpu.VMEM((2 * C_EDGE, 32), jnp.float32),
        pltpu.VMEM((DCH,), jnp.float32),
        pltpu.VMEM_SHARED((N,), jnp.float32),
        pltpu.VMEM_SHARED((N,), jnp.float32),
        pltpu.VMEM_SHARED((N, 32), jnp.float32),
        pltpu.VMEM_SHARED((N2,), jnp.float32),
    ] + [pltpu.SemaphoreType.DMA] * 12,
    compiler_params=pltpu.CompilerParams(needs_layout_passes=False, use_tc_tiling_on_sc=False),
)(_edge_body)


def _dot_body(agg_hbm, den_hbm, b_hbm, user_hbm, pos_hbm, out_hbm,
              uv, pv, uvh, pvh, rulo, ruhi, rplo, rphi, den_v, b_v, ov, sem):
    c = lax.axis_index("c")
    s = lax.axis_index("s")
    wid = c * NS + s

    pltpu.sync_copy(den_hbm, den_v)
    pltpu.sync_copy(b_hbm, b_v)
    pltpu.sync_copy(user_hbm.at[pl.ds(wid * 128, 128)], uv)
    pltpu.sync_copy(pos_hbm.at[pl.ds(wid * 128, 128)], pv)
    for i in range(8):
        pv[pl.ds(i * 16, 16)] = pv[pl.ds(i * 16, 16)] + N_USERS
        uvh[pl.ds(i * 16, 16)] = uv[pl.ds(i * 16, 16)] + N
    for i in range(8):
        pvh[pl.ds(i * 16, 16)] = pv[pl.ds(i * 16, 16)] + N

    d1 = pltpu.async_copy(agg_hbm.at[uv], rulo, sem)
    d2 = pltpu.async_copy(agg_hbm.at[pv], rplo, sem)
    d3 = pltpu.async_copy(agg_hbm.at[uvh], ruhi, sem)
    d4 = pltpu.async_copy(agg_hbm.at[pvh], rphi, sem)
    for d in (d1, d2, d3, d4):
        d.wait()

    bsegs = [b_v[pl.ds(i * 16, 16)] for i in range(4)]
    eps = jnp.float32(1e-16)

    @plsc.parallel_loop(0, 128, unroll=4)
    def _pair(r):
        u16 = plsc.load_gather(uv, [jnp.broadcast_to(r, (16,))])
        p16 = plsc.load_gather(pv, [jnp.broadcast_to(r, (16,))])
        du = jnp.float32(1.0) / (plsc.load_gather(den_v, [u16]) + eps)
        dp = jnp.float32(1.0) / (plsc.load_gather(den_v, [p16]) + eps)
        acc = jnp.zeros((16,), jnp.float32)
        for seg, ru, rp in ((0, rulo, rplo), (1, rulo, rplo),
                            (2, ruhi, rphi), (3, ruhi, rphi)):
            off = (seg % 2) * 16
            xu = ru[r, pl.ds(off, 16)] * du + bsegs[seg]
            xp = rp[r, pl.ds(off, 16)] * dp + bsegs[seg]
            acc = acc + xu * xp
        a = jnp.sum(acc, axis=0)
        plsc.store_scatter(ov, [jnp.broadcast_to(r, (16,))],
                           jnp.broadcast_to(a, (16,)),
                           mask=lax.iota(jnp.int32, 16) == 0)

    pltpu.sync_copy(ov, out_hbm.at[pl.ds(wid * 128, 128)])


_sc_dot = functools.partial(
    pl.kernel,
    out_type=jax.ShapeDtypeStruct((BATCH,), jnp.float32),
    mesh=_MESH,
    scratch_types=[
        pltpu.VMEM((128,), jnp.int32),
        pltpu.VMEM((128,), jnp.int32),
        pltpu.VMEM((128,), jnp.int32),
        pltpu.VMEM((128,), jnp.int32),
        pltpu.VMEM((128, 32), jnp.float32),
        pltpu.VMEM((128, 32), jnp.float32),
        pltpu.VMEM((128, 32), jnp.float32),
        pltpu.VMEM((128, 32), jnp.float32),
        pltpu.VMEM((N2,), jnp.float32),
        pltpu.VMEM((K,), jnp.float32),
        pltpu.VMEM((128,), jnp.float32),
        pltpu.SemaphoreType.DMA,
    ],
    compiler_params=pltpu.CompilerParams(needs_layout_passes=False, use_tc_tiling_on_sc=False),
)(_dot_body)


def _layer_glue(al):
    return al[:, 0], al[:, 1]


@jax.jit
def kernel(Gu, Gi, W0, a_src0, a_dst0, b0, W1, a_src1, a_dst1, b1,
           edge_index, user, pos):
    x0 = jnp.concatenate([Gu, Gi], axis=0)
    src2d = edge_index[0].astype(jnp.int32).reshape(E // 128, 128)
    dst2d = edge_index[1].astype(jnp.int32).reshape(E // 128, 128)
    user1 = user.astype(jnp.int32)
    pos1 = pos.astype(jnp.int32)

    def a2pad(a_src, a_dst, W):
        z = jnp.zeros((K, 62), jnp.float32)
        return jnp.concatenate(
            [(W @ a_src.reshape(K))[:, None],
             (W @ a_dst.reshape(K))[:, None], z], axis=1)

    def wsplit(W):
        return W.reshape(K, 2, 32).transpose(1, 0, 2)

    hlo, hhi, al = _tc_prep0(x0, wsplit(W0), a2pad(a_src0, a_dst0, W0))
    asv, adv = _layer_glue(al)
    agg, den = _sc_edge(asv, adv, src2d, dst2d, hlo, hhi)

    hlo, hhi, al = _tc_prep1(agg, den[:N].reshape(N, 1), b0.reshape(1, K),
                             wsplit(W1), a2pad(a_src1, a_dst1, W1))
    asv, adv = _layer_glue(al)
    agg, den = _sc_edge(asv, adv, src2d, dst2d, hlo, hhi)

    return _sc_dot(agg, den, b1, user1, pos1)
